# baseline (speedup 1.0000x reference)
import functools

import jax
import jax.numpy as jnp
from jax import lax
from jax.experimental import pallas as pl
from jax.experimental.pallas import tpu as pltpu
from jax.experimental.pallas import tpu_sc as plsc

_N = 10000
_E = 320000
_D_IN = 128
_CH = 16
_FCN = 64

_CHUNK = 128
_NCHUNKS = _E // _CHUNK
_ZB = 128
_NBUF = 8


def _chunk_range(wid, nw):
    c0 = (wid * _NCHUNKS) // nw
    c1 = ((wid + 1) * _NCHUNKS) // nw
    return c0, c1 - c0


def _deg_body(npad, ns, nc, rows_per_tile, k_tile, n_main,
              dst_hbm, out_hbm, didx, ones, zbuf, dacc,
              semi, *sems):
    cid = lax.axis_index("c")
    sid = lax.axis_index("s")
    wid = sid * nc + cid
    nw = nc * ns
    c0, n_w = _chunk_range(wid, nw)

    idx_desc = pltpu.async_copy(dst_hbm.at[1, pl.ds(c0, k_tile)], didx, semi)

    one = jnp.ones((16,), jnp.float32)
    zero = jnp.zeros((16,), jnp.float32)

    @pl.loop(0, _CHUNK)
    def _(r):
        ones[r, :] = one
        zbuf[r, :] = zero

    base_r = sid * rows_per_tile

    @pl.loop(0, rows_per_tile // _ZB)
    def _(i):
        pltpu.sync_copy(zbuf, dacc.at[pl.ds(base_r + i * _ZB, _ZB)])

    plsc.subcore_barrier()
    idx_desc.wait()

    @pl.loop(0, n_main // _NBUF)
    def _(t):
        for b in range(_NBUF):
            j = t * _NBUF + b

            @pl.when(t > 0)
            def _():
                pltpu.make_async_copy(ones, dacc.at[didx.at[j]],
                                      sems[b]).wait()

            pltpu.async_copy(ones, dacc.at[didx.at[j]], sems[b], add=True)

    for i in range(k_tile - n_main):
        j = n_main + i

        @pl.when(j < n_w)
        def _():
            pltpu.make_async_copy(ones, dacc.at[didx.at[j]],
                                  sems[i]).wait()
            pltpu.async_copy(ones, dacc.at[didx.at[j]], sems[i], add=True)

    for b in range(_NBUF):
        pltpu.make_async_copy(ones, dacc.at[didx.at[b]], sems[b]).wait()

    plsc.subcore_barrier()

    @pl.loop(0, rows_per_tile // _ZB)
    def _(i):
        r0 = base_r + i * _ZB
        pltpu.sync_copy(dacc.at[pl.ds(r0, _ZB)],
                        out_hbm.at[pl.ds(cid * npad + r0, _ZB)])


def _agg_body(npad, ns, nc, rows_per_tile, k_tile, n_main,
              ys_hbm, edge_hbm, out_hbm,
              sidx, didx, *scratch):
    cid = lax.axis_index("c")
    sid = lax.axis_index("s")
    wid = sid * nc + cid
    nw = nc * ns
    c0, n_w = _chunk_range(wid, nw)
    rows = scratch[:_NBUF]
    zbuf, acc, ysh, semi, semj, semy = scratch[_NBUF:_NBUF + 6]
    semg = scratch[_NBUF + 6:2 * _NBUF + 6]
    sems = scratch[2 * _NBUF + 6:]

    sdesc = pltpu.async_copy(edge_hbm.at[0, pl.ds(c0, k_tile)], sidx, semi)
    ddesc = pltpu.async_copy(edge_hbm.at[1, pl.ds(c0, k_tile)], didx, semj)

    base_r = sid * rows_per_tile
    ydesc = pltpu.async_copy(
        ys_hbm.at[pl.ds(base_r, rows_per_tile)],
        ysh.at[pl.ds(base_r, rows_per_tile)], semy)

    @pl.when(cid == 0)
    def _():
        pltpu.sync_copy(ys_hbm.at[pl.ds(base_r, rows_per_tile)],
                        acc.at[pl.ds(base_r, rows_per_tile)])

    @pl.when(cid != 0)
    def _():
        zero = jnp.zeros((16,), jnp.float32)

        @pl.loop(0, _ZB)
        def _(r):
            zbuf[r, :] = zero

        @pl.loop(0, rows_per_tile // _ZB)
        def _(i):
            pltpu.sync_copy(zbuf, acc.at[pl.ds(base_r + i * _ZB, _ZB)])

    ydesc.wait()
    plsc.subcore_barrier()
    sdesc.wait()
    ddesc.wait()

    def _gsrc(b):
        return ys_hbm if b % 4 == 3 else ysh

    @pl.loop(0, n_main // _NBUF)
    def _(t):
        for b in range(_NBUF):
            j = t * _NBUF + b

            @pl.when(t > 0)
            def _():
                pltpu.make_async_copy(rows[b], acc.at[didx.at[j]],
                                      sems[b]).wait()

            pltpu.async_copy(_gsrc(b).at[sidx.at[j]], rows[b], semg[b])

        for b in range(_NBUF):
            j = t * _NBUF + b
            pltpu.make_async_copy(_gsrc(b).at[sidx.at[j]], rows[b],
                                  semg[b]).wait()
            pltpu.async_copy(rows[b], acc.at[didx.at[j]], sems[b], add=True)

    for i in range(k_tile - n_main):
        j = n_main + i

        @pl.when(j < n_w)
        def _():
            pltpu.make_async_copy(rows[i], acc.at[didx.at[j]],
                                  sems[i]).wait()
            pltpu.async_copy(ysh.at[sidx.at[j]], rows[i], semg[i])

    for i in range(k_tile - n_main):
        j = n_main + i

        @pl.when(j < n_w)
        def _():
            pltpu.make_async_copy(ysh.at[sidx.at[j]], rows[i],
                                  semg[i]).wait()
            pltpu.async_copy(rows[i], acc.at[didx.at[j]], sems[i], add=True)

    for b in range(_NBUF):
        pltpu.make_async_copy(rows[b], acc.at[didx.at[b]], sems[b]).wait()

    plsc.subcore_barrier()

    @pl.loop(0, rows_per_tile // _ZB)
    def _(i):
        r0 = base_r + i * _ZB
        pltpu.sync_copy(acc.at[pl.ds(r0, _ZB)],
                        out_hbm.at[pl.ds(cid * npad + r0, _ZB)])


def _packed_dinv(npad, degp_ref):
    p = npad * _CH // 128
    d = degp_ref[0:p, :] + degp_ref[p:2 * p, :] + 1.0
    return lax.rsqrt(d)


def _mm_body(npad, x3_ref, wb_ref, o_ref):
    pr = _N * _CH // 128
    acc = jnp.zeros((pr, 128), jnp.float32)
    for i in range(8):
        acc = acc + jnp.dot(x3_ref[:, i, :], wb_ref[i],
                            preferred_element_type=jnp.float32)
    o_ref[0:pr, :] = acc
    o_ref[pr:npad * _CH // 128, :] = jnp.zeros(
        ((npad - _N) * _CH // 128, 128), jnp.float32)


def _scale_body(npad, degp_ref, xwp_ref, ysp_ref):
    ysp_ref[...] = xwp_ref[...] * _packed_dinv(npad, degp_ref)


def _head_body(npad, degp_ref, q_ref, w0_ref, b0_ref, w1_ref, b1_ref, o_ref):
    p = npad * _CH // 128
    h = jnp.maximum((q_ref[0:p, :] + q_ref[p:2 * p, :])
                    * _packed_dinv(npad, degp_ref), 0.0)
    o1 = jnp.dot(h, w0_ref[...], preferred_element_type=jnp.float32)
    o1 = jnp.maximum(o1 + b0_ref[...], 0.0)
    o_ref[...] = jnp.dot(o1, w1_ref[...],
                         preferred_element_type=jnp.float32) + b1_ref[...]


def kernel(x, edge_index, W_gcn, W0, b0, W1, b1):
    info = plsc.get_sparse_core_info()
    nc, ns = info.num_cores, info.num_subcores
    nw = nc * ns

    rows_per_tile = -(-_N // (ns * _ZB)) * _ZB
    npad = rows_per_tile * ns
    k_tile = -(-_NCHUNKS // nw)
    n_main = ((_NCHUNKS // nw) // _NBUF) * _NBUF
    assert k_tile - n_main <= _NBUF
    p = npad * _CH // 128

    edge3 = edge_index.astype(jnp.int32).reshape(2, _NCHUNKS, _CHUNK)

    mesh = plsc.VectorSubcoreMesh(core_axis_name="c", subcore_axis_name="s")
    sc_params = pltpu.CompilerParams(use_tc_tiling_on_sc=False)

    deg_kernel = functools.partial(
        pl.kernel,
        out_type=jax.ShapeDtypeStruct((nc * npad, _CH), jnp.float32),
        mesh=mesh,
        scratch_types=[
            pltpu.VMEM((k_tile, _CHUNK), jnp.int32),
            pltpu.VMEM((_CHUNK, _CH), jnp.float32),
            pltpu.VMEM((_ZB, _CH), jnp.float32),
            pltpu.VMEM_SHARED((npad, _CH), jnp.float32),
        ] + [pltpu.SemaphoreType.DMA] * (1 + _NBUF),
        compiler_params=sc_params,
    )(functools.partial(_deg_body, npad, ns, nc, rows_per_tile,
                        k_tile, n_main))

    agg_kernel = functools.partial(
        pl.kernel,
        out_type=jax.ShapeDtypeStruct((nc * npad, _CH), jnp.float32),
        mesh=mesh,
        scratch_types=[
            pltpu.VMEM((k_tile, _CHUNK), jnp.int32),
            pltpu.VMEM((k_tile, _CHUNK), jnp.int32),
        ] + [pltpu.VMEM((_CHUNK, _CH), jnp.float32)] * (_NBUF + 1) + [
            pltpu.VMEM_SHARED((npad, _CH), jnp.float32),
            pltpu.VMEM_SHARED((npad, _CH), jnp.float32),
        ] + [pltpu.SemaphoreType.DMA] * (3 + 2 * _NBUF),
        compiler_params=sc_params,
    )(functools.partial(_agg_body, npad, ns, nc, rows_per_tile,
                        k_tile, n_main))

    deg2 = deg_kernel(edge3)

    x3 = x.reshape(_N // 8, 8, _D_IN)
    wb = jnp.stack([jnp.pad(W_gcn, ((0, 0), (_CH * i, 128 - _CH * (i + 1))))
                    for i in range(8)])

    xwp = pl.pallas_call(
        functools.partial(_mm_body, npad),
        out_shape=jax.ShapeDtypeStruct((p, 128), jnp.float32),
    )(x3, wb)

    degp_v = deg2.reshape(nc * p, 128)

    ysp = pl.pallas_call(
        functools.partial(_scale_body, npad),
        out_shape=jax.ShapeDtypeStruct((p, 128), jnp.float32),
    )(degp_v, xwp)

    ys_lin = ysp.reshape(npad, _CH)
    agg2 = agg_kernel(ys_lin, edge3)

    q_v = agg2.reshape(nc * p, 128)
    w0blk = jnp.kron(jnp.eye(8, dtype=jnp.float32), W0)
    b0blk = jnp.tile(b0, 8).reshape(1, 8 * _FCN)
    w1blk = jnp.kron(jnp.eye(8, dtype=jnp.float32), W1)

    o2p = pl.pallas_call(
        functools.partial(_head_body, npad),
        out_shape=jax.ShapeDtypeStruct((p, 8), jnp.float32),
    )(degp_v, q_v, w0blk, b0blk, w1blk, b1.reshape(1, 1))

    return o2p.reshape(npad, 1)[:_N]


# speedup vs baseline: 1.0024x; 1.0024x over previous
import functools

import jax
import jax.numpy as jnp
from jax import lax
from jax.experimental import pallas as pl
from jax.experimental.pallas import tpu as pltpu
from jax.experimental.pallas import tpu_sc as plsc

_N = 10000
_E = 320000
_D_IN = 128
_CH = 16
_FCN = 64

_CHUNK = 128
_NCHUNKS = _E // _CHUNK
_ZB = 128
_NBUF = 8


def _chunk_range(wid, nw):
    c0 = (wid * _NCHUNKS) // nw
    c1 = ((wid + 1) * _NCHUNKS) // nw
    return c0, c1 - c0


def _deg_body(npad, ns, nc, rows_per_tile, k_tile, n_main,
              dst_hbm, out_hbm, didx, ones, zbuf, dacc,
              semi, *sems):
    cid = lax.axis_index("c")
    sid = lax.axis_index("s")
    wid = sid * nc + cid
    nw = nc * ns
    c0, n_w = _chunk_range(wid, nw)

    idx_desc = pltpu.async_copy(dst_hbm.at[1, pl.ds(c0, k_tile)], didx, semi)

    one = jnp.ones((16,), jnp.float32)
    zero = jnp.zeros((16,), jnp.float32)

    @pl.loop(0, _CHUNK)
    def _(r):
        ones[r, :] = one
        zbuf[r, :] = zero

    base_r = sid * rows_per_tile

    @pl.loop(0, rows_per_tile // _ZB)
    def _(i):
        pltpu.sync_copy(zbuf, dacc.at[pl.ds(base_r + i * _ZB, _ZB)])

    plsc.subcore_barrier()
    idx_desc.wait()

    @pl.loop(0, n_main // _NBUF)
    def _(t):
        for b in range(_NBUF):
            j = t * _NBUF + b

            @pl.when(t > 0)
            def _():
                pltpu.make_async_copy(ones, dacc.at[didx.at[j]],
                                      sems[b]).wait()

            pltpu.async_copy(ones, dacc.at[didx.at[j]], sems[b], add=True)

    for i in range(k_tile - n_main):
        j = n_main + i

        @pl.when(j < n_w)
        def _():
            pltpu.make_async_copy(ones, dacc.at[didx.at[j]],
                                  sems[i]).wait()
            pltpu.async_copy(ones, dacc.at[didx.at[j]], sems[i], add=True)

    for b in range(_NBUF):
        pltpu.make_async_copy(ones, dacc.at[didx.at[b]], sems[b]).wait()

    plsc.subcore_barrier()

    @pl.loop(0, rows_per_tile // _ZB)
    def _(i):
        r0 = base_r + i * _ZB
        pltpu.sync_copy(dacc.at[pl.ds(r0, _ZB)],
                        out_hbm.at[pl.ds(cid * npad + r0, _ZB)])


def _agg_body(npad, ns, nc, rows_per_tile, k_tile, n_main,
              ys_hbm, edge_hbm, out_hbm,
              sidx, didx, *scratch):
    cid = lax.axis_index("c")
    sid = lax.axis_index("s")
    wid = sid * nc + cid
    nw = nc * ns
    c0, n_w = _chunk_range(wid, nw)
    rows = scratch[:_NBUF]
    zbuf, acc, ysh, semi, semj, semy = scratch[_NBUF:_NBUF + 6]
    semg = scratch[_NBUF + 6:2 * _NBUF + 6]
    sems = scratch[2 * _NBUF + 6:]

    sdesc = pltpu.async_copy(edge_hbm.at[0, pl.ds(c0, k_tile)], sidx, semi)
    ddesc = pltpu.async_copy(edge_hbm.at[1, pl.ds(c0, k_tile)], didx, semj)

    base_r = sid * rows_per_tile
    ydesc = pltpu.async_copy(
        ys_hbm.at[pl.ds(base_r, rows_per_tile)],
        ysh.at[pl.ds(base_r, rows_per_tile)], semy)

    @pl.when(cid == 0)
    def _():
        pltpu.sync_copy(ys_hbm.at[pl.ds(base_r, rows_per_tile)],
                        acc.at[pl.ds(base_r, rows_per_tile)])

    @pl.when(cid != 0)
    def _():
        zero = jnp.zeros((16,), jnp.float32)

        @pl.loop(0, _ZB)
        def _(r):
            zbuf[r, :] = zero

        @pl.loop(0, rows_per_tile // _ZB)
        def _(i):
            pltpu.sync_copy(zbuf, acc.at[pl.ds(base_r + i * _ZB, _ZB)])

    ydesc.wait()
    plsc.subcore_barrier()
    sdesc.wait()
    ddesc.wait()

    def _gsrc(b):
        return ysh

    @pl.loop(0, n_main // _NBUF)
    def _(t):
        for b in range(_NBUF):
            j = t * _NBUF + b

            @pl.when(t > 0)
            def _():
                pltpu.make_async_copy(rows[b], acc.at[didx.at[j]],
                                      sems[b]).wait()

            pltpu.async_copy(_gsrc(b).at[sidx.at[j]], rows[b], semg[b])

        for b in range(_NBUF):
            j = t * _NBUF + b
            pltpu.make_async_copy(_gsrc(b).at[sidx.at[j]], rows[b],
                                  semg[b]).wait()
            pltpu.async_copy(rows[b], acc.at[didx.at[j]], sems[b], add=True)

    for i in range(k_tile - n_main):
        j = n_main + i

        @pl.when(j < n_w)
        def _():
            pltpu.make_async_copy(rows[i], acc.at[didx.at[j]],
                                  sems[i]).wait()
            pltpu.async_copy(ysh.at[sidx.at[j]], rows[i], semg[i])

    for i in range(k_tile - n_main):
        j = n_main + i

        @pl.when(j < n_w)
        def _():
            pltpu.make_async_copy(ysh.at[sidx.at[j]], rows[i],
                                  semg[i]).wait()
            pltpu.async_copy(rows[i], acc.at[didx.at[j]], sems[i], add=True)

    for b in range(_NBUF):
        pltpu.make_async_copy(rows[b], acc.at[didx.at[b]], sems[b]).wait()

    plsc.subcore_barrier()

    @pl.loop(0, rows_per_tile // _ZB)
    def _(i):
        r0 = base_r + i * _ZB
        pltpu.sync_copy(acc.at[pl.ds(r0, _ZB)],
                        out_hbm.at[pl.ds(cid * npad + r0, _ZB)])


def _packed_dinv(npad, degp_ref):
    p = npad * _CH // 128
    d = degp_ref[0:p, :] + degp_ref[p:2 * p, :] + 1.0
    return lax.rsqrt(d)


def _mm_body(npad, x3_ref, wb_ref, o_ref):
    pr = _N * _CH // 128
    acc = jnp.zeros((pr, 128), jnp.float32)
    for i in range(8):
        acc = acc + jnp.dot(x3_ref[:, i, :], wb_ref[i],
                            preferred_element_type=jnp.float32)
    o_ref[0:pr, :] = acc
    o_ref[pr:npad * _CH // 128, :] = jnp.zeros(
        ((npad - _N) * _CH // 128, 128), jnp.float32)


def _scale_body(npad, degp_ref, xwp_ref, ysp_ref):
    ysp_ref[...] = xwp_ref[...] * _packed_dinv(npad, degp_ref)


def _head_body(npad, degp_ref, q_ref, w0_ref, b0_ref, w1_ref, b1_ref, o_ref):
    p = npad * _CH // 128
    h = jnp.maximum((q_ref[0:p, :] + q_ref[p:2 * p, :])
                    * _packed_dinv(npad, degp_ref), 0.0)
    o1 = jnp.dot(h, w0_ref[...], preferred_element_type=jnp.float32)
    o1 = jnp.maximum(o1 + b0_ref[...], 0.0)
    o_ref[...] = jnp.dot(o1, w1_ref[...],
                         preferred_element_type=jnp.float32) + b1_ref[...]


def kernel(x, edge_index, W_gcn, W0, b0, W1, b1):
    info = plsc.get_sparse_core_info()
    nc, ns = info.num_cores, info.num_subcores
    nw = nc * ns

    rows_per_tile = -(-_N // (ns * _ZB)) * _ZB
    npad = rows_per_tile * ns
    k_tile = -(-_NCHUNKS // nw)
    n_main = ((_NCHUNKS // nw) // _NBUF) * _NBUF
    assert k_tile - n_main <= _NBUF
    p = npad * _CH // 128

    edge3 = edge_index.astype(jnp.int32).reshape(2, _NCHUNKS, _CHUNK)

    mesh = plsc.VectorSubcoreMesh(core_axis_name="c", subcore_axis_name="s")
    sc_params = pltpu.CompilerParams(use_tc_tiling_on_sc=False)

    deg_kernel = functools.partial(
        pl.kernel,
        out_type=jax.ShapeDtypeStruct((nc * npad, _CH), jnp.float32),
        mesh=mesh,
        scratch_types=[
            pltpu.VMEM((k_tile, _CHUNK), jnp.int32),
            pltpu.VMEM((_CHUNK, _CH), jnp.float32),
            pltpu.VMEM((_ZB, _CH), jnp.float32),
            pltpu.VMEM_SHARED((npad, _CH), jnp.float32),
        ] + [pltpu.SemaphoreType.DMA] * (1 + _NBUF),
        compiler_params=sc_params,
    )(functools.partial(_deg_body, npad, ns, nc, rows_per_tile,
                        k_tile, n_main))

    agg_kernel = functools.partial(
        pl.kernel,
        out_type=jax.ShapeDtypeStruct((nc * npad, _CH), jnp.float32),
        mesh=mesh,
        scratch_types=[
            pltpu.VMEM((k_tile, _CHUNK), jnp.int32),
            pltpu.VMEM((k_tile, _CHUNK), jnp.int32),
        ] + [pltpu.VMEM((_CHUNK, _CH), jnp.float32)] * (_NBUF + 1) + [
            pltpu.VMEM_SHARED((npad, _CH), jnp.float32),
            pltpu.VMEM_SHARED((npad, _CH), jnp.float32),
        ] + [pltpu.SemaphoreType.DMA] * (3 + 2 * _NBUF),
        compiler_params=sc_params,
    )(functools.partial(_agg_body, npad, ns, nc, rows_per_tile,
                        k_tile, n_main))

    deg2 = deg_kernel(edge3)

    x3 = x.reshape(_N // 8, 8, _D_IN)
    wb = jnp.stack([jnp.pad(W_gcn, ((0, 0), (_CH * i, 128 - _CH * (i + 1))))
                    for i in range(8)])

    xwp = pl.pallas_call(
        functools.partial(_mm_body, npad),
        out_shape=jax.ShapeDtypeStruct((p, 128), jnp.float32),
    )(x3, wb)

    degp_v = deg2.reshape(nc * p, 128)

    ysp = pl.pallas_call(
        functools.partial(_scale_body, npad),
        out_shape=jax.ShapeDtypeStruct((p, 128), jnp.float32),
    )(degp_v, xwp)

    ys_lin = ysp.reshape(npad, _CH)
    agg2 = agg_kernel(ys_lin, edge3)

    q_v = agg2.reshape(nc * p, 128)
    w0blk = jnp.kron(jnp.eye(8, dtype=jnp.float32), W0)
    b0blk = jnp.tile(b0, 8).reshape(1, 8 * _FCN)
    w1blk = jnp.kron(jnp.eye(8, dtype=jnp.float32), W1)

    o2p = pl.pallas_call(
        functools.partial(_head_body, npad),
        out_shape=jax.ShapeDtypeStruct((p, 8), jnp.float32),
    )(degp_v, q_v, w0blk, b0blk, w1blk, b1.reshape(1, 1))

    return o2p.reshape(npad, 1)[:_N]
